# baseline (speedup 1.0000x reference)
import functools

import jax
import jax.numpy as jnp
from jax import lax
from jax.experimental import pallas as pl
from jax.experimental.pallas import tpu as pltpu
from jax.experimental.pallas import tpu_sc as plsc

N = 10000
E = 320000
NF = 128
ND = 20
HARD_CUTOFF = 6.5

NW = 32
CHUNK = 128
PARTS = 5
EP = E // PARTS
NCHUNKS = EP // CHUNK
STEPS = (NCHUNKS + NW - 1) // NW

BB = 4000
NB = EP // BB

NFULL = N // CHUNK
NREM = N - NFULL * CHUNK


GG = 6
GSTEPS = (STEPS + GG - 1) // GG


def _sc_gather_body(x_hbm, ps_hbm, g_hbm,
                    i0, i1, i2, i3, i4, i5, r0, r1, r2, r3, r4, r5,
                    sem_i, sem_g, sem_w):
    wid = lax.axis_index("c") * 16 + lax.axis_index("s")
    idxs = [i0, i1, i2, i3, i4, i5]
    rows = [r0, r1, r2, r3, r4, r5]

    def grp(jj, carry):
        cs = [(jj * GG + b) * NW + wid for b in range(GG)]
        for b in range(GG):
            @pl.when(cs[b] < NCHUNKS)
            def _(b=b):
                pltpu.async_copy(ps_hbm.at[pl.ds(cs[b] * CHUNK, CHUNK)], idxs[b],
                                 sem_i)
        for b in range(GG):
            @pl.when(cs[b] < NCHUNKS)
            def _(b=b):
                pltpu.make_async_copy(ps_hbm.at[pl.ds(cs[b] * CHUNK, CHUNK)],
                                      idxs[b], sem_i).wait()
        for b in range(GG):
            @pl.when(cs[b] < NCHUNKS)
            def _(b=b):
                pltpu.async_copy(x_hbm.at[idxs[b]], rows[b], sem_g)
        for b in range(GG):
            @pl.when(cs[b] < NCHUNKS)
            def _(b=b):
                pltpu.make_async_copy(x_hbm.at[idxs[b]], rows[b], sem_g).wait()
        for b in range(GG):
            @pl.when(cs[b] < NCHUNKS)
            def _(b=b):
                pltpu.async_copy(
                    rows[b], g_hbm.at[pl.ds(cs[b] * CHUNK, CHUNK)], sem_w)
        for b in range(GG):
            @pl.when(cs[b] < NCHUNKS)
            def _(b=b):
                pltpu.make_async_copy(
                    rows[b], g_hbm.at[pl.ds(cs[b] * CHUNK, CHUNK)], sem_w).wait()
        return carry

    lax.fori_loop(0, GSTEPS, grp, 0)


def _sc_scatter_body(z_hbm, pf_hbm, out_hbm, idx_v, idx_v2, rows_v, rows_v2,
                     acc_sh, sem_z, sem_s):
    cid = lax.axis_index("c")
    sid = lax.axis_index("s")
    wid = cid * 16 + sid

    zeros16 = jnp.zeros((16,), jnp.float32)

    def zstep(i, carry):
        r = i // (NF // 16)
        col = (i % (NF // 16)) * 16
        rows_v[r, pl.ds(col, 16)] = zeros16
        return carry

    lax.fori_loop(0, CHUNK * (NF // 16), zstep, 0)

    for i in range((NFULL + 15) // 16):
        blk = sid + i * 16

        @pl.when(blk < NFULL)
        def _():
            pltpu.sync_copy(rows_v, acc_sh.at[pl.ds(blk * CHUNK, CHUNK)])

    @pl.when(sid == 0)
    def _():
        pltpu.sync_copy(rows_v.at[pl.ds(0, NREM)],
                        acc_sh.at[pl.ds(NFULL * CHUNK, NREM)])

    plsc.subcore_barrier()

    idxs = [idx_v, idx_v2]
    rows = [rows_v, rows_v2]

    def grp(jj, carry):
        cs = [(jj * 2 + b) * NW + wid for b in range(2)]
        for b in range(2):
            @pl.when(cs[b] < NCHUNKS)
            def _(b=b):
                pltpu.sync_copy(pf_hbm.at[pl.ds(cs[b] * CHUNK, CHUNK)], idxs[b])
                pltpu.async_copy(
                    z_hbm.at[pl.ds(cs[b] * CHUNK, CHUNK)], rows[b], sem_z)
        for b in range(2):
            @pl.when(cs[b] < NCHUNKS)
            def _(b=b):
                pltpu.make_async_copy(
                    z_hbm.at[pl.ds(cs[b] * CHUNK, CHUNK)], rows[b], sem_z).wait()
        for b in range(2):
            @pl.when(cs[b] < NCHUNKS)
            def _(b=b):
                pltpu.async_copy(rows[b], acc_sh.at[idxs[b]], sem_s, add=True)
        for b in range(2):
            @pl.when(cs[b] < NCHUNKS)
            def _(b=b):
                pltpu.make_async_copy(
                    rows[b], acc_sh.at[idxs[b]], sem_s).wait()
        return carry

    lax.fori_loop(0, (STEPS + 1) // 2, grp, 0)
    plsc.subcore_barrier()

    for i in range((NFULL + 15) // 16):
        blk = sid + i * 16

        @pl.when(blk < NFULL)
        def _():
            pltpu.sync_copy(acc_sh.at[pl.ds(blk * CHUNK, CHUNK)], rows_v)
            pltpu.sync_copy(rows_v, out_hbm.at[pl.ds(cid * N + blk * CHUNK, CHUNK)])

    @pl.when(sid == 0)
    def _():
        pltpu.sync_copy(acc_sh.at[pl.ds(NFULL * CHUNK, NREM)],
                        rows_v.at[pl.ds(0, NREM)])
        pltpu.sync_copy(rows_v.at[pl.ds(0, NREM)],
                        out_hbm.at[pl.ds(cid * N + NFULL * CHUNK, NREM)])


def _tc_z_body(g_ref, d_ref, w_ref, mu_ref, sg_ref, z_ref):
    gt = jnp.transpose(g_ref[...])
    d = d_ref[0]
    inv = 1.0 / d
    cut = jnp.where(
        d < HARD_CUTOFF,
        jnp.cos(d * (jnp.pi / (2.0 * HARD_CUTOFF))) ** 2,
        0.0,
    )
    rows = []
    for k in range(ND):
        t = (inv - mu_ref[0, k]) / sg_ref[0, k]
        rows.append(jnp.exp(-0.5 * t * t) * cut)
    kr = jnp.concatenate([rows[k] * gt for k in range(ND)], axis=0)
    zt = jnp.dot(w_ref[...], kr, preferred_element_type=jnp.float32)
    z_ref[...] = jnp.transpose(zt)


TD = 400


def _tc_out_body(p1_ref, p2_ref, p3_ref, p4_ref, p5_ref, x_ref, w_ref, b_ref,
                 o_ref):
    s = jnp.dot(x_ref[...], w_ref[...], preferred_element_type=jnp.float32)
    o_ref[...] = (((p1_ref[0] + p1_ref[1]) + (p2_ref[0] + p2_ref[1]))
                  + ((p3_ref[0] + p3_ref[1]) + (p4_ref[0] + p4_ref[1]))
                  + (p5_ref[0] + p5_ref[1]) + s + b_ref[...])


def kernel(in_features, pair_first, pair_second, dist_pairs, mu, sigma,
           int_weights, self_W, self_b):
    ps = pair_second.astype(jnp.int32)
    pf = pair_first.astype(jnp.int32)
    x = in_features.astype(jnp.float32)

    mesh = plsc.VectorSubcoreMesh(core_axis_name="c", subcore_axis_name="s")

    gather = pl.kernel(
        _sc_gather_body,
        out_type=jax.ShapeDtypeStruct((EP, NF), jnp.float32),
        mesh=mesh,
        scratch_types=(
            [pltpu.VMEM((CHUNK,), jnp.int32)] * GG
            + [pltpu.VMEM((CHUNK, NF), jnp.float32)] * GG
            + [pltpu.SemaphoreType.DMA] * 3
        ),
    )

    wm = jnp.transpose(int_weights, (1, 0, 2)).reshape(NF, ND * NF)
    mu2 = mu.astype(jnp.float32).reshape(1, ND)
    sg2 = sigma.astype(jnp.float32).reshape(1, ND)

    def tc_z(g_part, dist_part):
        return pl.pallas_call(
            _tc_z_body,
            grid=(NB,),
            in_specs=[
                pl.BlockSpec((BB, NF), lambda b: (b, 0)),
                pl.BlockSpec((1, 1, BB), lambda b: (b, 0, 0)),
                pl.BlockSpec((NF, ND * NF), lambda b: (0, 0)),
                pl.BlockSpec(memory_space=pltpu.SMEM),
                pl.BlockSpec(memory_space=pltpu.SMEM),
            ],
            out_specs=pl.BlockSpec((BB, NF), lambda b: (b, 0)),
            out_shape=jax.ShapeDtypeStruct((EP, NF), jnp.float32),
        )(g_part, dist_part.reshape(NB, 1, BB), wm, mu2, sg2)

    scatter = pl.kernel(
        _sc_scatter_body,
        out_type=jax.ShapeDtypeStruct((2 * N, NF), jnp.float32),
        mesh=mesh,
        scratch_types=[
            pltpu.VMEM((CHUNK,), jnp.int32),
            pltpu.VMEM((CHUNK,), jnp.int32),
            pltpu.VMEM((CHUNK, NF), jnp.float32),
            pltpu.VMEM((CHUNK, NF), jnp.float32),
            pltpu.VMEM_SHARED((N, NF), jnp.float32),
            pltpu.SemaphoreType.DMA,
            pltpu.SemaphoreType.DMA,
        ],
    )

    dist = dist_pairs.astype(jnp.float32)
    gs = [gather(x, ps[i * EP:(i + 1) * EP]) for i in range(PARTS)]
    zs = [tc_z(gs[i], dist[i * EP:(i + 1) * EP]) for i in range(PARTS)]
    pps = [scatter(zs[i], pf[i * EP:(i + 1) * EP]).reshape(2, N, NF)
           for i in range(PARTS)]

    swt = jnp.transpose(self_W, (1, 0)).astype(jnp.float32)
    b2 = self_b.astype(jnp.float32).reshape(1, NF)
    out = pl.pallas_call(
        _tc_out_body,
        grid=(N // TD,),
        in_specs=[
            pl.BlockSpec((2, TD, NF), lambda b: (0, b, 0)),
            pl.BlockSpec((2, TD, NF), lambda b: (0, b, 0)),
            pl.BlockSpec((2, TD, NF), lambda b: (0, b, 0)),
            pl.BlockSpec((2, TD, NF), lambda b: (0, b, 0)),
            pl.BlockSpec((2, TD, NF), lambda b: (0, b, 0)),
            pl.BlockSpec((TD, NF), lambda b: (b, 0)),
            pl.BlockSpec((NF, NF), lambda b: (0, 0)),
            pl.BlockSpec((1, NF), lambda b: (0, 0)),
        ],
        out_specs=pl.BlockSpec((TD, NF), lambda b: (b, 0)),
        out_shape=jax.ShapeDtypeStruct((N, NF), jnp.float32),
    )(pps[0], pps[1], pps[2], pps[3], pps[4], x, swt, b2)
    return out
